# baseline (speedup 1.0000x reference)
import functools

import jax
import jax.numpy as jnp
from jax import lax
from jax.experimental import pallas as pl
from jax.experimental.pallas import tpu as pltpu
from jax.experimental.pallas import tpu_sc as plsc

N = 10000
D = 128
H = 128
K = 5000
N_PAD = 10240

_BR = 1024
_KP = 5008
_G = 8
_NS = 16
_NW = 32
_WMAIN = 9984
_WPAD = 10112


def _scores_body(x_ref, w1_ref, b1_ref, w2_ref, b2_ref, out_ref):
    xb = x_ref[...]
    h = jnp.tanh(jnp.dot(xb, w1_ref[...], preferred_element_type=jnp.float32) + b1_ref[...])
    logit = jnp.dot(h, w2_ref[...], preferred_element_type=jnp.float32) + b2_ref[...]
    out_ref[...] = logit


def _scores(x_pad, W1, b1, W2, b2):
    BR = 1024
    return pl.pallas_call(
        _scores_body,
        grid=(N_PAD // BR,),
        in_specs=[
            pl.BlockSpec((BR, D), lambda i: (i, 0)),
            pl.BlockSpec((D, H), lambda i: (0, 0)),
            pl.BlockSpec((1, H), lambda i: (0, 0)),
            pl.BlockSpec((H, 1), lambda i: (0, 0)),
            pl.BlockSpec((1, 1), lambda i: (0, 0)),
        ],
        out_specs=pl.BlockSpec((BR, 1), lambda i: (i, 0)),
        out_shape=jax.ShapeDtypeStruct((N_PAD, 1), jnp.float32),
    )(x_pad, W1, b1.reshape(1, H), W2, b2.reshape(1, 1))


def _rank_body(scol_ref, srow_ref, out_ref):
    pid = pl.program_id(0)
    ig = lax.broadcasted_iota(jnp.int32, (_BR, 1), 0) + pid * _BR
    si = jnp.where(ig < N, scol_ref[...], -1.0)
    jg0 = lax.broadcasted_iota(jnp.int32, (1, _BR), 1)
    acc = jnp.zeros((_BR, 1), jnp.int32)
    for jb in range(N_PAD // _BR):
        jg = jg0 + jb * _BR
        sj = jnp.where(jg < N, srow_ref[pl.ds(jb, 1), :], -1.0)
        contrib = (sj > si) | ((jb < pid) & (sj >= si))
        acc = acc + jnp.sum(contrib.astype(jnp.int32), axis=1, keepdims=True)
    jgd = jg0 + pid * _BR
    sjd = jnp.where(jgd < N, srow_ref[pl.ds(pid, 1), :], -1.0)
    tie = (sjd == si) & (jg0 < ig - pid * _BR)
    acc = acc + jnp.sum(tie.astype(jnp.int32), axis=1, keepdims=True)
    out_ref[...] = acc


def _ranks(scores_col, scores_row):
    return pl.pallas_call(
        _rank_body,
        grid=(N_PAD // _BR,),
        in_specs=[
            pl.BlockSpec((_BR, 1), lambda i: (i, 0)),
            pl.BlockSpec((N_PAD // _BR, _BR), lambda i: (0, 0)),
        ],
        out_specs=pl.BlockSpec((_BR, 1), lambda i: (i, 0)),
        out_shape=jax.ShapeDtypeStruct((N_PAD, 1), jnp.int32),
    )(scores_col, scores_row)


def _sc_pool(ranks1, adj, tail, x):
    mesh = plsc.VectorSubcoreMesh(core_axis_name="c", subcore_axis_name="s")

    @functools.partial(
        pl.kernel,
        out_type=(jax.ShapeDtypeStruct((K * K,), jnp.float32),
                  jax.ShapeDtypeStruct((K, D), jnp.float32)),
        mesh=mesh,
        compiler_params=pltpu.CompilerParams(needs_layout_passes=False),
        scratch_types=[
            pltpu.VMEM((640,), jnp.int32),
            pltpu.VMEM((640,), jnp.int32),
            pltpu.VMEM((_KP,), jnp.int32),
            pltpu.VMEM((_G, _WPAD), jnp.float32),
            pltpu.VMEM((_G * _KP,), jnp.float32),
            pltpu.VMEM((16, D), jnp.float32),
            pltpu.VMEM_SHARED((N_PAD,), jnp.int32),
            pltpu.SemaphoreType.DMA,
            pltpu.SemaphoreType.DMA,
        ],
    )
    def k(ranks_hbm, adj_hbm, tail_hbm, x_hbm, adjp_hbm, xp_hbm,
          ranks_v, vals_v, idx_v, octb, outb_v, xb,
          idx_shared, sem_in, sem_out):
        cid = lax.axis_index("c")
        sid = lax.axis_index("s")
        wid = cid * _NS + sid

        pltpu.sync_copy(ranks_hbm.at[pl.ds(sid * 640, 640)], ranks_v)
        base = sid * 640
        for j in range(40):
            vals_v[pl.ds(j * 16, 16)] = lax.iota(jnp.int32, 16) + (base + j * 16)
        for j in range(40):
            rvec = ranks_v[pl.ds(j * 16, 16)]
            pltpu.async_copy(vals_v.at[pl.ds(j * 16, 16)],
                             idx_shared.at[rvec], sem_out)
        for j in range(40):
            rvec = ranks_v[pl.ds(j * 16, 16)]
            pltpu.make_async_copy(vals_v.at[pl.ds(j * 16, 16)],
                                  idx_shared.at[rvec], sem_out).wait()
        plsc.subcore_barrier()

        pltpu.sync_copy(idx_shared.at[pl.ds(0, _KP)], idx_v)
        r0 = pl.multiple_of(
            jnp.where(wid < 17, wid * 160, 2720 + (wid - 17) * 152), 8)
        cnt = jnp.where(wid < 17, 160, 152)

        rowids = [jnp.full((16,), r, jnp.int32) for r in range(_G)]

        def group_body(g, _):
            b = pl.multiple_of(r0 + g * _G, 8)
            gidx = idx_v.at[pl.ds(b, _G)]
            pltpu.async_copy(adj_hbm.at[gidx, pl.ds(0, _WMAIN)],
                             octb.at[:, pl.ds(0, _WMAIN)], sem_in)
            pltpu.async_copy(tail_hbm.at[gidx],
                             octb.at[:, pl.ds(_WMAIN, 128)], sem_in)
            @pl.when(g > 0)
            def _():
                bp = pl.multiple_of(r0 + (g - 1) * _G, 8)
                for r in range(_G):
                    dstp = pl.multiple_of((bp + r) * K, 8)
                    pltpu.make_async_copy(outb_v.at[pl.ds(r * _KP, K)],
                                          adjp_hbm.at[pl.ds(dstp, K)],
                                          sem_out).wait()
            pltpu.make_async_copy(adj_hbm.at[gidx, pl.ds(0, _WMAIN)],
                                  octb.at[:, pl.ds(0, _WMAIN)], sem_in).wait()
            pltpu.make_async_copy(tail_hbm.at[gidx],
                                  octb.at[:, pl.ds(_WMAIN, 128)], sem_in).wait()

            @plsc.parallel_loop(0, _KP // 16, 1, unroll=2)
            def col_body(c):
                c16 = pl.multiple_of(c * 16, 16)
                colv = idx_v[pl.ds(c16, 16)]
                for r in range(_G):
                    outb_v[pl.ds(r * _KP + c16, 16)] = plsc.load_gather(
                        octb, [rowids[r], colv])
            for r in range(_G):
                dst = pl.multiple_of((b + r) * K, 8)
                pltpu.async_copy(outb_v.at[pl.ds(r * _KP, K)],
                                 adjp_hbm.at[pl.ds(dst, K)], sem_out)
            return 0

        lax.fori_loop(0, cnt // _G, group_body, 0)
        blast = pl.multiple_of(r0 + cnt - _G, 8)
        for r in range(_G):
            dstl = pl.multiple_of((blast + r) * K, 8)
            pltpu.make_async_copy(outb_v.at[pl.ds(r * _KP, K)],
                                  adjp_hbm.at[pl.ds(dstl, K)], sem_out).wait()

        def x_body(h2, _):
            bx = pl.multiple_of(r0 + jnp.minimum(h2 * 16, cnt - 16), 8)
            pltpu.async_copy(x_hbm.at[idx_v.at[pl.ds(bx, 16)]], xb,
                             sem_in).wait()
            pltpu.sync_copy(xb, xp_hbm.at[pl.ds(bx, 16)])
            return 0

        lax.fori_loop(0, 10, x_body, 0)

    return k(ranks1, adj, tail, x)


def _proj_body(xp_ref, wp_ref, bp_ref, out_ref, acc_ref):
    pid = pl.program_id(0)
    y = jnp.maximum(
        jnp.dot(xp_ref[...], wp_ref[...], preferred_element_type=jnp.float32)
        + bp_ref[...], 0.0)
    out_ref[...] = y

    @pl.when(pid == 0)
    def _():
        acc_ref[...] = jnp.zeros_like(acc_ref)

    acc_ref[...] += jnp.sum(y, axis=0, keepdims=True)


def _project(xp, Wp, bp):
    BRP = 1000
    return pl.pallas_call(
        _proj_body,
        grid=(K // BRP,),
        in_specs=[
            pl.BlockSpec((BRP, D), lambda i: (i, 0)),
            pl.BlockSpec((D, H), lambda i: (0, 0)),
            pl.BlockSpec((1, H), lambda i: (0, 0)),
        ],
        out_specs=[
            pl.BlockSpec((BRP, H), lambda i: (i, 0)),
            pl.BlockSpec((1, H), lambda i: (0, 0)),
        ],
        out_shape=[
            jax.ShapeDtypeStruct((K, H), jnp.float32),
            jax.ShapeDtypeStruct((1, H), jnp.float32),
        ],
    )(xp, Wp, bp.reshape(1, H))


def kernel(x, adj, W1, b1, W2, b2, Wp, bp):
    x_pad = jnp.pad(x, ((0, N_PAD - N), (0, 0)))
    logits = _scores(x_pad, W1, b1, W2, b2)
    scores = jax.nn.sigmoid(logits)
    ranks = _ranks(scores, scores.reshape(N_PAD // _BR, _BR))
    tail = jnp.pad(adj[:, _WMAIN:], ((0, 0), (0, 128 - (N - _WMAIN))))
    adj_flat, x_pooled = _sc_pool(ranks.reshape(N_PAD), adj, tail, x)
    adj_pooled = adj_flat.reshape(K, K)
    x_projected, colsum = _project(x_pooled, Wp, bp)
    graph_rep = (colsum * (1.0 / K)).reshape(H)
    return (x_projected, adj_pooled, graph_rep)


# speedup vs baseline: 1.0107x; 1.0107x over previous
import functools

import jax
import jax.numpy as jnp
from jax import lax
from jax.experimental import pallas as pl
from jax.experimental.pallas import tpu as pltpu
from jax.experimental.pallas import tpu_sc as plsc

N = 10000
D = 128
H = 128
K = 5000
N_PAD = 10240

_BR = 1024
_KP = 5008
_G = 8
_NS = 16
_NW = 32
_WMAIN = 9984
_WPAD = 10112


def _scores_body(x_ref, w1_ref, b1_ref, w2_ref, b2_ref, out_ref):
    xb = x_ref[...]
    h = jnp.tanh(jnp.dot(xb, w1_ref[...], preferred_element_type=jnp.float32) + b1_ref[...])
    logit = jnp.dot(h, w2_ref[...], preferred_element_type=jnp.float32) + b2_ref[...]
    out_ref[...] = logit


def _scores(x_pad, W1, b1, W2, b2):
    BR = 1024
    return pl.pallas_call(
        _scores_body,
        grid=(N_PAD // BR,),
        in_specs=[
            pl.BlockSpec((BR, D), lambda i: (i, 0)),
            pl.BlockSpec((D, H), lambda i: (0, 0)),
            pl.BlockSpec((1, H), lambda i: (0, 0)),
            pl.BlockSpec((H, 1), lambda i: (0, 0)),
            pl.BlockSpec((1, 1), lambda i: (0, 0)),
        ],
        out_specs=pl.BlockSpec((BR, 1), lambda i: (i, 0)),
        out_shape=jax.ShapeDtypeStruct((N_PAD, 1), jnp.float32),
    )(x_pad, W1, b1.reshape(1, H), W2, b2.reshape(1, 1))


def _rank_body(scol_ref, srow_ref, out_ref):
    pid = pl.program_id(0)
    ig = lax.broadcasted_iota(jnp.int32, (_BR, 1), 0) + pid * _BR
    si = jnp.where(ig < N, scol_ref[...], -1.0)
    jg0 = lax.broadcasted_iota(jnp.int32, (1, _BR), 1)
    acc = jnp.zeros((_BR, 1), jnp.int32)
    for jb in range(N_PAD // _BR):
        jg = jg0 + jb * _BR
        sj = jnp.where(jg < N, srow_ref[pl.ds(jb, 1), :], -1.0)
        contrib = (sj > si) | ((sj == si) & (jg < ig))
        acc = acc + jnp.sum(contrib.astype(jnp.int32), axis=1, keepdims=True)
    out_ref[...] = acc


def _ranks(scores_col, scores_row):
    return pl.pallas_call(
        _rank_body,
        grid=(N_PAD // _BR,),
        in_specs=[
            pl.BlockSpec((_BR, 1), lambda i: (i, 0)),
            pl.BlockSpec((N_PAD // _BR, _BR), lambda i: (0, 0)),
        ],
        out_specs=pl.BlockSpec((_BR, 1), lambda i: (i, 0)),
        out_shape=jax.ShapeDtypeStruct((N_PAD, 1), jnp.int32),
    )(scores_col, scores_row)


def _sc_pool(ranks1, adj, tail, x):
    mesh = plsc.VectorSubcoreMesh(core_axis_name="c", subcore_axis_name="s")

    @functools.partial(
        pl.kernel,
        out_type=(jax.ShapeDtypeStruct((K * K,), jnp.float32),
                  jax.ShapeDtypeStruct((K, D), jnp.float32)),
        mesh=mesh,
        compiler_params=pltpu.CompilerParams(needs_layout_passes=False),
        scratch_types=[
            pltpu.VMEM((640,), jnp.int32),
            pltpu.VMEM((640,), jnp.int32),
            pltpu.VMEM((_KP,), jnp.int32),
            pltpu.VMEM((_G, _WPAD), jnp.float32),
            pltpu.VMEM((_G * _KP,), jnp.float32),
            pltpu.VMEM((16, D), jnp.float32),
            pltpu.VMEM_SHARED((N_PAD,), jnp.int32),
            pltpu.SemaphoreType.DMA,
            pltpu.SemaphoreType.DMA,
        ],
    )
    def k(ranks_hbm, adj_hbm, tail_hbm, x_hbm, adjp_hbm, xp_hbm,
          ranks_v, vals_v, idx_v, octb, outb_v, xb,
          idx_shared, sem_in, sem_out):
        cid = lax.axis_index("c")
        sid = lax.axis_index("s")
        wid = cid * _NS + sid

        pltpu.sync_copy(ranks_hbm.at[pl.ds(sid * 640, 640)], ranks_v)
        base = sid * 640
        for j in range(40):
            vals_v[pl.ds(j * 16, 16)] = lax.iota(jnp.int32, 16) + (base + j * 16)
        for j in range(40):
            rvec = ranks_v[pl.ds(j * 16, 16)]
            pltpu.async_copy(vals_v.at[pl.ds(j * 16, 16)],
                             idx_shared.at[rvec], sem_out)
        for j in range(40):
            rvec = ranks_v[pl.ds(j * 16, 16)]
            pltpu.make_async_copy(vals_v.at[pl.ds(j * 16, 16)],
                                  idx_shared.at[rvec], sem_out).wait()
        plsc.subcore_barrier()

        pltpu.sync_copy(idx_shared.at[pl.ds(0, _KP)], idx_v)
        r0 = pl.multiple_of(
            jnp.where(wid < 17, wid * 160, 2720 + (wid - 17) * 152), 8)
        cnt = jnp.where(wid < 17, 160, 152)

        rowids = [jnp.full((16,), r, jnp.int32) for r in range(_G)]

        def group_body(g, _):
            b = pl.multiple_of(r0 + g * _G, 8)
            gidx = idx_v.at[pl.ds(b, _G)]
            pltpu.async_copy(adj_hbm.at[gidx, pl.ds(0, _WMAIN)],
                             octb.at[:, pl.ds(0, _WMAIN)], sem_in)
            pltpu.async_copy(tail_hbm.at[gidx],
                             octb.at[:, pl.ds(_WMAIN, 128)], sem_in)
            @pl.when(g > 0)
            def _():
                bp = pl.multiple_of(r0 + (g - 1) * _G, 8)
                for r in range(_G):
                    dstp = pl.multiple_of((bp + r) * K, 8)
                    pltpu.make_async_copy(outb_v.at[pl.ds(r * _KP, K)],
                                          adjp_hbm.at[pl.ds(dstp, K)],
                                          sem_out).wait()
            pltpu.make_async_copy(adj_hbm.at[gidx, pl.ds(0, _WMAIN)],
                                  octb.at[:, pl.ds(0, _WMAIN)], sem_in).wait()
            pltpu.make_async_copy(tail_hbm.at[gidx],
                                  octb.at[:, pl.ds(_WMAIN, 128)], sem_in).wait()

            @plsc.parallel_loop(0, _KP // 16, 1, unroll=2)
            def col_body(c):
                c16 = pl.multiple_of(c * 16, 16)
                colv = idx_v[pl.ds(c16, 16)]
                for r in range(_G):
                    outb_v[pl.ds(r * _KP + c16, 16)] = plsc.load_gather(
                        octb, [rowids[r], colv])
            for r in range(_G):
                dst = pl.multiple_of((b + r) * K, 8)
                pltpu.async_copy(outb_v.at[pl.ds(r * _KP, K)],
                                 adjp_hbm.at[pl.ds(dst, K)], sem_out)
            return 0

        lax.fori_loop(0, cnt // _G, group_body, 0)
        blast = pl.multiple_of(r0 + cnt - _G, 8)
        for r in range(_G):
            dstl = pl.multiple_of((blast + r) * K, 8)
            pltpu.make_async_copy(outb_v.at[pl.ds(r * _KP, K)],
                                  adjp_hbm.at[pl.ds(dstl, K)], sem_out).wait()

        def x_body(h2, _):
            bx = pl.multiple_of(r0 + jnp.minimum(h2 * 16, cnt - 16), 8)
            pltpu.async_copy(x_hbm.at[idx_v.at[pl.ds(bx, 16)]], xb,
                             sem_in).wait()
            pltpu.sync_copy(xb, xp_hbm.at[pl.ds(bx, 16)])
            return 0

        lax.fori_loop(0, 10, x_body, 0)

    return k(ranks1, adj, tail, x)


def _proj_body(xp_ref, wp_ref, bp_ref, out_ref, acc_ref):
    pid = pl.program_id(0)
    y = jnp.maximum(
        jnp.dot(xp_ref[...], wp_ref[...], preferred_element_type=jnp.float32)
        + bp_ref[...], 0.0)
    out_ref[...] = y

    @pl.when(pid == 0)
    def _():
        acc_ref[...] = jnp.zeros_like(acc_ref)

    acc_ref[...] += jnp.sum(y, axis=0, keepdims=True)


def _project(xp, Wp, bp):
    BRP = 1000
    return pl.pallas_call(
        _proj_body,
        grid=(K // BRP,),
        in_specs=[
            pl.BlockSpec((BRP, D), lambda i: (i, 0)),
            pl.BlockSpec((D, H), lambda i: (0, 0)),
            pl.BlockSpec((1, H), lambda i: (0, 0)),
        ],
        out_specs=[
            pl.BlockSpec((BRP, H), lambda i: (i, 0)),
            pl.BlockSpec((1, H), lambda i: (0, 0)),
        ],
        out_shape=[
            jax.ShapeDtypeStruct((K, H), jnp.float32),
            jax.ShapeDtypeStruct((1, H), jnp.float32),
        ],
    )(xp, Wp, bp.reshape(1, H))


def kernel(x, adj, W1, b1, W2, b2, Wp, bp):
    x_pad = jnp.pad(x, ((0, N_PAD - N), (0, 0)))
    logits = _scores(x_pad, W1, b1, W2, b2)
    scores = jax.nn.sigmoid(logits)
    ranks = _ranks(scores, scores.reshape(N_PAD // _BR, _BR))
    tail = jnp.pad(adj[:, _WMAIN:], ((0, 0), (0, 128 - (N - _WMAIN))))
    adj_flat, x_pooled = _sc_pool(ranks.reshape(N_PAD), adj, tail, x)
    adj_pooled = adj_flat.reshape(K, K)
    x_projected, colsum = _project(x_pooled, Wp, bp)
    graph_rep = (colsum * (1.0 / K)).reshape(H)
    return (x_projected, adj_pooled, graph_rep)
